# Initial kernel scaffold; baseline (speedup 1.0000x reference)
import jax
import jax.numpy as jnp
from jax.experimental import pallas as pl


def kernel(logits, labels):
    raise NotImplementedError("write your pallas kernel here")


# speedup vs baseline: 1.2966x; 1.2966x over previous
import functools

import jax
import jax.numpy as jnp
from jax import lax
from jax.experimental import pallas as pl
from jax.experimental.pallas import tpu as pltpu
from jax.experimental.pallas import tpu_sc as plsc

N_BINS = 10
N_ROWS = 16384
N_CLASSES = 1000

ROW_BLK = 1024


def _rowstats_body(x_ref, lab_ref, v_ref, seg_ref):
    x = x_ref[...]
    m = jnp.max(x, axis=1, keepdims=True)
    col = lax.broadcasted_iota(jnp.int32, x.shape, 1)
    pred = jnp.min(jnp.where(x == m, col, N_CLASSES), axis=1, keepdims=True)
    sumexp = jnp.sum(jnp.exp(x - m), axis=1, keepdims=True)
    conf = 1.0 / sumexp
    bin_idx = jnp.clip(
        jnp.ceil(conf * N_BINS).astype(jnp.int32) - 1, 0, N_BINS - 1)
    acc = (pred == lab_ref[...]).astype(jnp.float32)
    v_ref[...] = conf - acc
    seg_ref[...] = pred * N_BINS + bin_idx


def _rowstats(logits, labels2d):
    grid = N_ROWS // ROW_BLK
    return pl.pallas_call(
        _rowstats_body,
        grid=(grid,),
        in_specs=[
            pl.BlockSpec((ROW_BLK, N_CLASSES), lambda i: (i, 0)),
            pl.BlockSpec((ROW_BLK, 1), lambda i: (i, 0)),
        ],
        out_specs=[
            pl.BlockSpec((ROW_BLK, 1), lambda i: (i, 0)),
            pl.BlockSpec((ROW_BLK, 1), lambda i: (i, 0)),
        ],
        out_shape=[
            jax.ShapeDtypeStruct((N_ROWS, 1), jnp.float32),
            jax.ShapeDtypeStruct((N_ROWS, 1), jnp.int32),
        ],
    )(logits, labels2d)


HSZ_PER_TILE = 640
HSZ = 16 * HSZ_PER_TILE
CHUNK = N_ROWS // 16


def _histogram_ece(v, seg):
    mesh = plsc.VectorSubcoreMesh(core_axis_name="c", subcore_axis_name="s")

    @functools.partial(
        pl.kernel,
        mesh=mesh,
        out_type=jax.ShapeDtypeStruct((16,), jnp.float32),
        scratch_types=[
            pltpu.VMEM((CHUNK,), jnp.float32),
            pltpu.VMEM((CHUNK,), jnp.int32),
            pltpu.VMEM((HSZ_PER_TILE,), jnp.float32),
            pltpu.VMEM((16,), jnp.float32),
            pltpu.VMEM((256,), jnp.float32),
            pltpu.VMEM((16,), jnp.int32),
            pltpu.VMEM_SHARED((HSZ,), jnp.float32),
            pltpu.VMEM_SHARED((256,), jnp.float32),
        ],
    )
    def sck(v_hbm, seg_hbm, out_hbm, v_v, seg_v, h_v, acc_v, pacc_v, zidx_v,
            hist_sh, part_sh):
        cid = lax.axis_index("c")
        sid = lax.axis_index("s")

        @pl.when(cid == 0)
        def _core0():
            zero16 = jnp.zeros((16,), jnp.float32)
            for i in range(HSZ_PER_TILE // 16):
                h_v[pl.ds(i * 16, 16)] = zero16
            pltpu.sync_copy(h_v, hist_sh.at[pl.ds(sid * HSZ_PER_TILE,
                                                  HSZ_PER_TILE)])
            plsc.subcore_barrier()

            base = sid * CHUNK
            pltpu.sync_copy(v_hbm.at[pl.ds(base, CHUNK)], v_v)
            pltpu.sync_copy(seg_hbm.at[pl.ds(base, CHUNK)], seg_v)
            pltpu.sync_copy(v_v, hist_sh.at[seg_v], add=True)
            plsc.subcore_barrier()

            pltpu.sync_copy(hist_sh.at[pl.ds(sid * HSZ_PER_TILE,
                                             HSZ_PER_TILE)], h_v)
            acc = jnp.zeros((16,), jnp.float32)
            for i in range(HSZ_PER_TILE // 16):
                acc = acc + jnp.abs(h_v[pl.ds(i * 16, 16)])
            acc_v[...] = acc
            pltpu.sync_copy(acc_v, part_sh.at[pl.ds(sid * 16, 16)])
            plsc.subcore_barrier()

            @pl.when(sid == 0)
            def _tile0():
                pltpu.sync_copy(part_sh, pacc_v)
                tot = jnp.zeros((16,), jnp.float32)
                for t in range(16):
                    tot = tot + pacc_v[pl.ds(t * 16, 16)]
                acc_v[...] = tot
                zidx_v[...] = jnp.full((16,), HSZ - 16, jnp.int32)
                pltpu.sync_copy(acc_v, hist_sh.at[zidx_v], add=True)
                pltpu.sync_copy(hist_sh.at[pl.ds(HSZ - 16, 16)], acc_v)
                acc_v[...] = acc_v[...] * (1.0 / N_ROWS)
                pltpu.sync_copy(acc_v, out_hbm)

    return sck(v, seg)


def kernel(logits, labels):
    labels2d = labels.reshape(N_ROWS, 1)
    v, seg = _rowstats(logits, labels2d)
    out = _histogram_ece(v.reshape(N_ROWS), seg.reshape(N_ROWS))
    return out[0:1]


# speedup vs baseline: 1.5028x; 1.1590x over previous
import functools

import jax
import jax.numpy as jnp
from jax import lax
from jax.experimental import pallas as pl
from jax.experimental.pallas import tpu as pltpu
from jax.experimental.pallas import tpu_sc as plsc

N_BINS = 10
N_ROWS = 16384
N_CLASSES = 1000

CHUNK_R = 512
N_CHUNKS = N_ROWS // CHUNK_R
NBUF = 4


def _rowstats_body(x_hbm, conf_ref, pred_ref, buf, cscr, pscr, sems):
    def copy(i, slot):
        return pltpu.make_async_copy(
            x_hbm.at[pl.ds(i * CHUNK_R, CHUNK_R), :], buf.at[slot],
            sems.at[slot])

    for k in range(NBUF):
        copy(k, k).start()

    for i in range(N_CHUNKS):
        slot = i % NBUF
        copy(i, slot).wait()
        x = buf[slot]
        m = jnp.max(x, axis=1, keepdims=True)
        col = lax.broadcasted_iota(jnp.int32, x.shape, 1)
        pred = jnp.min(jnp.where(x == m, col, N_CLASSES), axis=1,
                       keepdims=True)
        sumexp = jnp.sum(jnp.exp(x - m), axis=1, keepdims=True)
        cscr[:, i:i + 1] = 1.0 / sumexp
        pscr[:, i:i + 1] = pred
        if i + NBUF < N_CHUNKS:
            copy(i + NBUF, slot).start()

    conf_ref[...] = cscr[...].T
    pred_ref[...] = pscr[...].T


def _rowstats(logits):
    return pl.pallas_call(
        _rowstats_body,
        in_specs=[pl.BlockSpec(memory_space=pl.ANY)],
        out_specs=[
            pl.BlockSpec((N_CHUNKS, CHUNK_R), lambda: (0, 0)),
            pl.BlockSpec((N_CHUNKS, CHUNK_R), lambda: (0, 0)),
        ],
        out_shape=[
            jax.ShapeDtypeStruct((N_CHUNKS, CHUNK_R), jnp.float32),
            jax.ShapeDtypeStruct((N_CHUNKS, CHUNK_R), jnp.int32),
        ],
        scratch_shapes=[
            pltpu.VMEM((NBUF, CHUNK_R, N_CLASSES), jnp.float32),
            pltpu.VMEM((CHUNK_R, N_CHUNKS), jnp.float32),
            pltpu.VMEM((CHUNK_R, N_CHUNKS), jnp.int32),
            pltpu.SemaphoreType.DMA((NBUF,)),
        ],
    )(logits)


HSZ_PER_TILE = 640
HSZ = 16 * HSZ_PER_TILE
CHUNK = N_ROWS // 16


def _histogram_ece(conf, pred, labels):
    mesh = plsc.VectorSubcoreMesh(core_axis_name="c", subcore_axis_name="s")

    @functools.partial(
        pl.kernel,
        mesh=mesh,
        out_type=jax.ShapeDtypeStruct((16,), jnp.float32),
        scratch_types=[
            pltpu.VMEM((CHUNK,), jnp.float32),
            pltpu.VMEM((CHUNK,), jnp.int32),
            pltpu.VMEM((CHUNK,), jnp.int32),
            pltpu.VMEM((HSZ_PER_TILE,), jnp.float32),
            pltpu.VMEM((16,), jnp.float32),
            pltpu.VMEM((256,), jnp.float32),
            pltpu.VMEM((16,), jnp.int32),
            pltpu.VMEM_SHARED((HSZ,), jnp.float32),
            pltpu.VMEM_SHARED((256,), jnp.float32),
        ],
    )
    def sck(conf_hbm, pred_hbm, lab_hbm, out_hbm, v_v, seg_v, lab_v, h_v,
            acc_v, pacc_v, zidx_v, hist_sh, part_sh):
        cid = lax.axis_index("c")
        sid = lax.axis_index("s")

        @pl.when(cid == 0)
        def _core0():
            zero16 = jnp.zeros((16,), jnp.float32)
            for i in range(HSZ_PER_TILE // 16):
                h_v[pl.ds(i * 16, 16)] = zero16
            pltpu.sync_copy(h_v, hist_sh.at[pl.ds(sid * HSZ_PER_TILE,
                                                  HSZ_PER_TILE)])
            plsc.subcore_barrier()

            base = sid * CHUNK
            pltpu.sync_copy(conf_hbm.at[pl.ds(base, CHUNK)], v_v)
            pltpu.sync_copy(pred_hbm.at[pl.ds(base, CHUNK)], seg_v)
            pltpu.sync_copy(lab_hbm.at[pl.ds(base, CHUNK)], lab_v)
            for j in range(CHUNK // 16):
                sl = pl.ds(j * 16, 16)
                c = v_v[sl]
                p = seg_v[sl]
                l = lab_v[sl]
                acc = jnp.where(p == l, 1.0, 0.0)
                t = c * float(N_BINS)
                it = t.astype(jnp.int32)
                b = jnp.where(it.astype(jnp.float32) == t, it - 1, it)
                b = jnp.clip(b, 0, N_BINS - 1)
                v_v[sl] = c - acc
                seg_v[sl] = p * N_BINS + b
            pltpu.sync_copy(v_v, hist_sh.at[seg_v], add=True)
            plsc.subcore_barrier()

            pltpu.sync_copy(hist_sh.at[pl.ds(sid * HSZ_PER_TILE,
                                             HSZ_PER_TILE)], h_v)
            acc = jnp.zeros((16,), jnp.float32)
            for i in range(HSZ_PER_TILE // 16):
                acc = acc + jnp.abs(h_v[pl.ds(i * 16, 16)])
            acc_v[...] = acc
            pltpu.sync_copy(acc_v, part_sh.at[pl.ds(sid * 16, 16)])
            plsc.subcore_barrier()

            @pl.when(sid == 0)
            def _tile0():
                pltpu.sync_copy(part_sh, pacc_v)
                tot = jnp.zeros((16,), jnp.float32)
                for t in range(16):
                    tot = tot + pacc_v[pl.ds(t * 16, 16)]
                acc_v[...] = tot
                zidx_v[...] = jnp.full((16,), HSZ - 16, jnp.int32)
                pltpu.sync_copy(acc_v, hist_sh.at[zidx_v], add=True)
                pltpu.sync_copy(hist_sh.at[pl.ds(HSZ - 16, 16)], acc_v)
                acc_v[...] = acc_v[...] * (1.0 / N_ROWS)
                pltpu.sync_copy(acc_v, out_hbm)

    return sck(conf, pred, labels)


def kernel(logits, labels):
    conf, pred = _rowstats(logits)
    out = _histogram_ece(conf.reshape(N_ROWS), pred.reshape(N_ROWS), labels)
    return out[0:1]
